# baseline (speedup 1.0000x reference)
import functools

import jax
import jax.numpy as jnp
from jax import lax
from jax.experimental import pallas as pl
from jax.experimental.pallas import tpu as pltpu
from jax.experimental.pallas import tpu_sc as plsc

N = 10000
E = 320000
D_IN = 128
D_OUT = 128
NC = 2
NS = 16
L = 16
HALF = D_OUT // NC
CH = 128
NCHUNK = E // CH
ROWS_PER_SUB = N // NS


_TC_BLK = 1000


def _tc_linear_body(x_ref, w_ref, b_ref, st_ref):
    res = lax.dot_general(
        x_ref[...], w_ref[...],
        dimension_numbers=(((1,), (1,)), ((), ())),
        preferred_element_type=jnp.float32,
    )
    res = res + b_ref[...]
    st_ref[0] = res[:, :HALF]
    st_ref[1] = res[:, HALF:]


def _tc_linear(x, W, b2):
    return pl.pallas_call(
        _tc_linear_body,
        grid=(N // _TC_BLK,),
        in_specs=[
            pl.BlockSpec((_TC_BLK, D_IN), lambda i: (i, 0)),
            pl.BlockSpec((D_OUT, D_IN), lambda i: (0, 0)),
            pl.BlockSpec((1, D_OUT), lambda i: (0, 0)),
        ],
        out_specs=pl.BlockSpec((NC, _TC_BLK, HALF), lambda i: (0, i, 0)),
        out_shape=jax.ShapeDtypeStruct((NC, N, HALF), jnp.float32),
    )(x, W, b2)


_sc_mesh = plsc.VectorSubcoreMesh(core_axis_name="c", subcore_axis_name="s")

_GATHER_DNUMS = lax.GatherDimensionNumbers(
    offset_dims=(), collapsed_slice_dims=(0,), start_index_map=(0,))

NG = CH // L
B = 4
SB = 39
MAIN = SB * B


@functools.partial(
    pl.kernel,
    out_type=jax.ShapeDtypeStruct((N, D_OUT), jnp.float32),
    mesh=_sc_mesh,
    scratch_types=[
        pltpu.VMEM((2, B, CH), jnp.int32),
        pltpu.VMEM((2, 2, B, CH), jnp.int32),
        pltpu.VMEM((2, B, CH), jnp.float32),
        pltpu.VMEM((2, B, CH, HALF), jnp.float32),
        pltpu.VMEM_SHARED((N, HALF), jnp.float32),
        pltpu.SemaphoreType.DMA,
        pltpu.SemaphoreType.DMA,
        pltpu.SemaphoreType.DMA,
        pltpu.SemaphoreType.DMA,
        pltpu.SemaphoreType.DMA,
    ],
    compiler_params=pltpu.CompilerParams(use_tc_tiling_on_sc=False),
)
def _sc_spmm(table, ei, ew, zeros, out, srcb, dstb, wb, rows, agg,
             semA, semB, ssemA, ssemB, isem):
    c = lax.axis_index("c")
    s = lax.axis_index("s")

    pltpu.sync_copy(zeros.at[pl.ds(s * ROWS_PER_SUB, ROWS_PER_SUB)],
                    agg.at[pl.ds(s * ROWS_PER_SUB, ROWS_PER_SUB)])
    plsc.subcore_barrier()

    row_off = c * N
    r0 = s * MAIN

    def load_idx(t, p, h):
        base = r0 + t * B
        d1 = pltpu.async_copy(ei.at[1, pl.ds(base, B)], srcb.at[p], isem)
        d2 = pltpu.async_copy(ei.at[0, pl.ds(base, B)], dstb.at[p, h], isem)
        d3 = pltpu.async_copy(ew.at[pl.ds(base, B)], wb.at[p], isem)
        d1.wait()
        d2.wait()
        d3.wait()
        for j in range(B):
            for k in range(NG):
                srcb[p, j, pl.ds(k * L, L)] = (
                    srcb[p, j, pl.ds(k * L, L)] + row_off)

    def fire(p, sem):
        for j in range(B):
            pltpu.async_copy(table.at[srcb.at[p, j]], rows.at[p, j], sem)

    def drain(p, sem):
        for j in range(B):
            pltpu.make_async_copy(
                table.at[srcb.at[p, j]], rows.at[p, j], sem).wait()

    def scale_group(p, j, g):
        wv = wb[p, j, pl.ds(g * L, L)]
        for l in range(L):
            wj = lax.gather(
                wv, jnp.full((L, 1), l, jnp.int32), _GATHER_DNUMS,
                (1,), mode=lax.GatherScatterMode.PROMISE_IN_BOUNDS)
            e = g * L + l
            for k in range(HALF // L):
                rows[p, j, e, pl.ds(k * L, L)] = (
                    rows[p, j, e, pl.ds(k * L, L)] * wj)

    def scale(p):
        @plsc.parallel_loop(0, B * NG, unroll=4)
        def _(gi):
            scale_group(p, gi >> 3, gi & (NG - 1))

    def fire_scatter(p, h, sem):
        for j in range(B):
            pltpu.make_async_copy(
                rows.at[p, j], agg.at[dstb.at[p, h, j]], sem).start(add=True)

    def drain_scatter(p, h, sem):
        for j in range(B):
            pltpu.make_async_copy(
                rows.at[p, j], agg.at[dstb.at[p, h, j]], sem).wait()

    load_idx(0, 0, 0)
    fire(0, semA)

    def super_body(i, carry):
        t0 = 2 * i
        h = i % 2
        load_idx(t0 + 1, 1, h)

        @pl.when(i > 0)
        def _():
            drain_scatter(1, (i - 1) % 2, ssemB)

        fire(1, semB)
        drain(0, semA)
        scale(0)
        fire_scatter(0, h, ssemA)
        load_idx(t0 + 2, 0, (i + 1) % 2)
        drain_scatter(0, h, ssemA)
        fire(0, semA)
        drain(1, semB)
        scale(1)
        fire_scatter(1, h, ssemB)
        return carry

    lax.fori_loop(0, SB // 2, super_body, 0)
    drain_scatter(1, (SB // 2 - 1) % 2, ssemB)
    drain(0, semA)
    scale(0)
    fire_scatter(0, (SB // 2) % 2, ssemA)
    drain_scatter(0, (SB // 2) % 2, ssemA)

    @pl.when(s < 4)
    def _():
        tc = SB * B * NS + s
        pltpu.sync_copy(ei.at[1, tc], srcb.at[0, 0])
        pltpu.sync_copy(ei.at[0, tc], dstb.at[0, 0, 0])
        pltpu.sync_copy(ew.at[tc], wb.at[0, 0])
        for k in range(NG):
            srcb[0, 0, pl.ds(k * L, L)] = (
                srcb[0, 0, pl.ds(k * L, L)] + row_off)
        pltpu.async_copy(table.at[srcb.at[0, 0]], rows.at[0, 0], semA).wait()

        def gbody(g, carry):
            scale_group(0, 0, g)
            return carry
        lax.fori_loop(0, NG, gbody, 0)
        pltpu.sync_copy(rows.at[0, 0], agg.at[dstb.at[0, 0, 0]], add=True)

    plsc.subcore_barrier()

    pltpu.sync_copy(
        agg.at[pl.ds(s * ROWS_PER_SUB, ROWS_PER_SUB)],
        out.at[pl.ds(s * ROWS_PER_SUB, ROWS_PER_SUB), pl.ds(c * HALF, HALF)],
    )


@jax.jit
def _impl(x, edge_index, edge_weight, W, b):
    st = _tc_linear(x, W, b.reshape(1, D_OUT))
    table = st.reshape(NC * N, HALF)
    ei = edge_index.reshape(2, NCHUNK, CH)
    ew = edge_weight.reshape(NCHUNK, CH)
    zeros = jnp.zeros((N, HALF), jnp.float32)
    return _sc_spmm(table, ei, ew, zeros)


def kernel(x, edge_index, edge_weight, W, b):
    return _impl(x, edge_index, edge_weight, W, b)


# speedup vs baseline: 1.0807x; 1.0807x over previous
import functools

import jax
import jax.numpy as jnp
from jax import lax
from jax.experimental import pallas as pl
from jax.experimental.pallas import tpu as pltpu
from jax.experimental.pallas import tpu_sc as plsc

N = 10000
E = 320000
D_IN = 128
D_OUT = 128
NC = 2
NS = 16
L = 16
HALF = D_OUT // NC
CH = 128
NCHUNK = E // CH
ROWS_PER_SUB = N // NS


_TC_BLK = 1000


def _tc_linear_body(x_ref, w_ref, b_ref, st_ref):
    res = lax.dot_general(
        x_ref[...], w_ref[...],
        dimension_numbers=(((1,), (1,)), ((), ())),
        preferred_element_type=jnp.float32,
    )
    res = res + b_ref[...]
    st_ref[0] = res[:, :HALF]
    st_ref[1] = res[:, HALF:]


def _tc_linear(x, W, b2):
    return pl.pallas_call(
        _tc_linear_body,
        grid=(N // _TC_BLK,),
        in_specs=[
            pl.BlockSpec((_TC_BLK, D_IN), lambda i: (i, 0)),
            pl.BlockSpec((D_OUT, D_IN), lambda i: (0, 0)),
            pl.BlockSpec((1, D_OUT), lambda i: (0, 0)),
        ],
        out_specs=pl.BlockSpec((NC, _TC_BLK, HALF), lambda i: (0, i, 0)),
        out_shape=jax.ShapeDtypeStruct((NC, N, HALF), jnp.float32),
    )(x, W, b2)


_sc_mesh = plsc.VectorSubcoreMesh(core_axis_name="c", subcore_axis_name="s")

_GATHER_DNUMS = lax.GatherDimensionNumbers(
    offset_dims=(), collapsed_slice_dims=(0,), start_index_map=(0,))

NG = CH // L
B = 4
SB = 39
MAIN = SB * B


@functools.partial(
    pl.kernel,
    out_type=jax.ShapeDtypeStruct((N, D_OUT), jnp.float32),
    mesh=_sc_mesh,
    scratch_types=[
        pltpu.VMEM((2, B, CH), jnp.int32),
        pltpu.VMEM((2, 2, B, CH), jnp.int32),
        pltpu.VMEM((2, B, CH), jnp.float32),
        pltpu.VMEM((2, B, CH, HALF), jnp.float32),
        pltpu.VMEM_SHARED((N, HALF), jnp.float32),
        pltpu.SemaphoreType.DMA,
        pltpu.SemaphoreType.DMA,
        pltpu.SemaphoreType.DMA,
        pltpu.SemaphoreType.DMA,
        pltpu.SemaphoreType.DMA,
    ],
    compiler_params=pltpu.CompilerParams(use_tc_tiling_on_sc=False),
)
def _sc_spmm(table, ei, ew, zeros, out, srcb, dstb, wb, rows, agg,
             semA, semB, ssemA, ssemB, isem):
    c = lax.axis_index("c")
    s = lax.axis_index("s")

    pltpu.sync_copy(zeros.at[pl.ds(s * ROWS_PER_SUB, ROWS_PER_SUB)],
                    agg.at[pl.ds(s * ROWS_PER_SUB, ROWS_PER_SUB)])
    plsc.subcore_barrier()

    row_off = c * N
    r0 = s * MAIN

    def load_idx(t, p, h):
        base = r0 + t * B
        d1 = pltpu.async_copy(ei.at[1, pl.ds(base, B)], srcb.at[p], isem)
        d2 = pltpu.async_copy(ei.at[0, pl.ds(base, B)], dstb.at[p, h], isem)
        d3 = pltpu.async_copy(ew.at[pl.ds(base, B)], wb.at[p], isem)
        d1.wait()
        d2.wait()
        d3.wait()
        for j in range(B):
            for k in range(NG):
                srcb[p, j, pl.ds(k * L, L)] = (
                    srcb[p, j, pl.ds(k * L, L)] + row_off)

    def fire(p, sem):
        for j in range(B):
            pltpu.async_copy(table.at[srcb.at[p, j]], rows.at[p, j], sem)

    def drain(p, sem):
        for j in range(B):
            pltpu.make_async_copy(
                table.at[srcb.at[p, j]], rows.at[p, j], sem).wait()

    def scale_group(p, j, g):
        wv = wb[p, j, pl.ds(g * L, L)]
        for l in range(L):
            wj = lax.gather(
                wv, jnp.full((L, 1), l, jnp.int32), _GATHER_DNUMS,
                (1,), mode=lax.GatherScatterMode.PROMISE_IN_BOUNDS)
            e = g * L + l
            for k in range(HALF // L):
                rows[p, j, e, pl.ds(k * L, L)] = (
                    rows[p, j, e, pl.ds(k * L, L)] * wj)

    def scale(p):
        @plsc.parallel_loop(0, B * NG, unroll=2)
        def _(gi):
            scale_group(p, gi >> 3, gi & (NG - 1))

    def fire_scatter(p, h, sem):
        for j in range(B):
            pltpu.make_async_copy(
                rows.at[p, j], agg.at[dstb.at[p, h, j]], sem).start(add=True)

    def drain_scatter(p, h, sem):
        for j in range(B):
            pltpu.make_async_copy(
                rows.at[p, j], agg.at[dstb.at[p, h, j]], sem).wait()

    load_idx(0, 0, 0)
    fire(0, semA)

    def super_body(i, carry):
        t0 = 2 * i
        h = i % 2
        load_idx(t0 + 1, 1, h)

        @pl.when(i > 0)
        def _():
            drain_scatter(1, (i - 1) % 2, ssemB)

        fire(1, semB)
        drain(0, semA)
        scale(0)
        fire_scatter(0, h, ssemA)
        load_idx(t0 + 2, 0, (i + 1) % 2)
        drain_scatter(0, h, ssemA)
        fire(0, semA)
        drain(1, semB)
        scale(1)
        fire_scatter(1, h, ssemB)
        return carry

    lax.fori_loop(0, SB // 2, super_body, 0)
    drain_scatter(1, (SB // 2 - 1) % 2, ssemB)
    drain(0, semA)
    scale(0)
    fire_scatter(0, (SB // 2) % 2, ssemA)
    drain_scatter(0, (SB // 2) % 2, ssemA)

    @pl.when(s < 4)
    def _():
        tc = SB * B * NS + s
        pltpu.sync_copy(ei.at[1, tc], srcb.at[0, 0])
        pltpu.sync_copy(ei.at[0, tc], dstb.at[0, 0, 0])
        pltpu.sync_copy(ew.at[tc], wb.at[0, 0])
        for k in range(NG):
            srcb[0, 0, pl.ds(k * L, L)] = (
                srcb[0, 0, pl.ds(k * L, L)] + row_off)
        pltpu.async_copy(table.at[srcb.at[0, 0]], rows.at[0, 0], semA).wait()

        def gbody(g, carry):
            scale_group(0, 0, g)
            return carry
        lax.fori_loop(0, NG, gbody, 0)
        pltpu.sync_copy(rows.at[0, 0], agg.at[dstb.at[0, 0, 0]], add=True)

    plsc.subcore_barrier()

    pltpu.sync_copy(
        agg.at[pl.ds(s * ROWS_PER_SUB, ROWS_PER_SUB)],
        out.at[pl.ds(s * ROWS_PER_SUB, ROWS_PER_SUB), pl.ds(c * HALF, HALF)],
    )


@jax.jit
def _impl(x, edge_index, edge_weight, W, b):
    st = _tc_linear(x, W, b.reshape(1, D_OUT))
    table = st.reshape(NC * N, HALF)
    ei = edge_index.reshape(2, NCHUNK, CH)
    ew = edge_weight.reshape(NCHUNK, CH)
    zeros = jnp.zeros((N, HALF), jnp.float32)
    return _sc_spmm(table, ei, ew, zeros)


def kernel(x, edge_index, edge_weight, W, b):
    return _impl(x, edge_index, edge_weight, W, b)
